# baseline (speedup 1.0000x reference)
import jax
import jax.numpy as jnp
from jax.experimental import pallas as pl

_MARGIN = 1.0
_B = 1024
_T = 128
_NT = _B // _T
_NPAIRS = _B * (_B - 1) // 2
_CONTRACT_LAST = (((1,), (1,)), ((), ()))


def _loss_kernel(emb_ref, lab_ref, out_ref):
    e = emb_ref[:]
    lab = lab_ref[:]

    m = jnp.max(lab, axis=1, keepdims=True)
    col = jax.lax.broadcasted_iota(jnp.int32, (1, lab.shape[1]), 1)
    w = jax.lax.bitcast_convert_type((127 - col) << 23, jnp.float32)
    mw = jnp.where(lab == m, w, 0.0)
    z = jax.lax.dot_general(mw, jnp.ones((1, lab.shape[1]), jnp.float32),
                            _CONTRACT_LAST,
                            preferred_element_type=jnp.float32)
    zbits = jax.lax.bitcast_convert_type(z, jnp.int32)
    idx_f = (127 - (zbits >> 23)).astype(jnp.float32)

    one = jnp.ones((1, 1), jnp.float32)
    idx_row = jax.lax.dot_general(one, idx_f, _CONTRACT_LAST,
                                  preferred_element_type=jnp.float32)

    n_vec = jnp.sum(e * e, axis=1, keepdims=True)
    ones_col = jnp.ones((_B, 1), jnp.float32)
    a_aug = jnp.concatenate([-2.0 * e, n_vec, ones_col], axis=1)
    b_aug = jnp.concatenate([e, ones_col, n_vec], axis=1)

    def d2_tile(bi, bj):
        return jnp.maximum(
            jax.lax.dot_general(a_aug[bi * _T:(bi + 1) * _T, :],
                                b_aug[bj * _T:(bj + 1) * _T, :],
                                _CONTRACT_LAST,
                                preferred_element_type=jnp.float32), 1e-12)

    def eq_tile(bi, bj):
        return idx_f[bi * _T:(bi + 1) * _T, :] == idx_row[:, bj * _T:(bj + 1) * _T]

    acc_off = jnp.zeros((_T, _T), jnp.float32)
    acc_diag = jnp.zeros((_T, _T), jnp.float32)
    mn = jnp.full((_T, _T), jnp.inf, jnp.float32)
    for bi in range(_NT):
        for bj in range(bi, _NT):
            d2 = d2_tile(bi, bj)
            mn = jnp.minimum(mn, d2)
            pos = jnp.where(eq_tile(bi, bj), d2, 0.0)
            if bi == bj:
                acc_diag = acc_diag + pos
            else:
                acc_off = acc_off + pos

    def neg_sum():
        noff = jnp.zeros((_T, _T), jnp.float32)
        ndiag = jnp.zeros((_T, _T), jnp.float32)
        for bi in range(_NT):
            for bj in range(bi, _NT):
                d2 = d2_tile(bi, bj)
                s = d2 * jax.lax.rsqrt(d2)
                t = jnp.maximum(_MARGIN - s, 0.0)
                neg = jnp.where(eq_tile(bi, bj), 0.0, t * t)
                if bi == bj:
                    ndiag = ndiag + neg
                else:
                    noff = noff + neg
        ntot = noff + 0.5 * ndiag
        return jnp.sum(ntot, axis=1, keepdims=True).sum(axis=0, keepdims=True)

    gmin = jnp.min(mn)
    neg_total = jax.lax.cond(gmin < _MARGIN * _MARGIN * 1.01,
                             neg_sum,
                             lambda: jnp.zeros((1, 1), jnp.float32))

    tot = acc_off + 0.5 * acc_diag
    row_sums = jnp.sum(tot, axis=1, keepdims=True)
    total = jnp.sum(row_sums, axis=0, keepdims=True) + neg_total
    out_ref[:, :] = total / _NPAIRS


def kernel(embeddings, labels):
    out = pl.pallas_call(
        _loss_kernel,
        out_shape=jax.ShapeDtypeStruct((1, 1), jnp.float32),
    )(embeddings, labels)
    return out[0, 0]


# speedup vs baseline: 1.1467x; 1.1467x over previous
import jax
import jax.numpy as jnp
from jax.experimental import pallas as pl

_MARGIN = 1.0
_B = 1024
_T = 128
_NT = _B // _T
_NPAIRS = _B * (_B - 1) // 2
_CONTRACT_LAST = (((1,), (1,)), ((), ()))


def _loss_kernel(emb_ref, lab_ref, out_ref):
    e = emb_ref[:]
    lab = lab_ref[:]

    m = jnp.max(lab, axis=1, keepdims=True)
    col = jax.lax.broadcasted_iota(jnp.int32, (1, lab.shape[1]), 1)
    w = jax.lax.bitcast_convert_type((127 - col) << 23, jnp.float32)
    mw = jnp.where(lab == m, w, 0.0)
    z = jax.lax.dot_general(mw, jnp.ones((1, lab.shape[1]), jnp.float32),
                            _CONTRACT_LAST,
                            preferred_element_type=jnp.float32)
    zbits = jax.lax.bitcast_convert_type(z, jnp.int32)
    idx_f = (127 - (zbits >> 23)).astype(jnp.float32)

    one = jnp.ones((1, 1), jnp.float32)
    idx_row = jax.lax.dot_general(one, idx_f, _CONTRACT_LAST,
                                  preferred_element_type=jnp.float32)

    n_vec = jnp.sum(e * e, axis=1, keepdims=True)
    ones_col = jnp.ones((_B, 1), jnp.float32)
    a_aug = jnp.concatenate([-2.0 * e, n_vec, ones_col], axis=1)
    b_aug = jnp.concatenate([e, ones_col, n_vec], axis=1)

    acc_off = jnp.zeros((_T, _T), jnp.float32)
    acc_diag = jnp.zeros((_T, _T), jnp.float32)
    for bi in range(_NT):
        a_blk = a_aug[bi * _T:(bi + 1) * _T, :]
        idc = idx_f[bi * _T:(bi + 1) * _T, :]
        for bj in range(bi, _NT):
            b_blk = b_aug[bj * _T:(bj + 1) * _T, :]
            idr = idx_row[:, bj * _T:(bj + 1) * _T]
            d2 = jnp.maximum(
                jax.lax.dot_general(a_blk, b_blk, _CONTRACT_LAST,
                                    preferred_element_type=jnp.float32), 1e-12)
            s = d2 * jax.lax.rsqrt(d2)
            t = jnp.maximum(_MARGIN - s, 0.0)
            loss_t = jnp.where(idc == idr, d2, t * t)
            if bi == bj:
                acc_diag = acc_diag + loss_t
            else:
                acc_off = acc_off + loss_t
    tot = acc_off + 0.5 * acc_diag
    row_sums = jnp.sum(tot, axis=1, keepdims=True)
    total = jnp.sum(row_sums, axis=0, keepdims=True)
    out_ref[:, :] = total / _NPAIRS


def kernel(embeddings, labels):
    out = pl.pallas_call(
        _loss_kernel,
        out_shape=jax.ShapeDtypeStruct((1, 1), jnp.float32),
    )(embeddings, labels)
    return out[0, 0]
